# baseline (speedup 1.0000x reference)
import functools

import jax
import jax.numpy as jnp
from jax import lax
from jax.experimental import pallas as pl
from jax.experimental.pallas import tpu as pltpu
from jax.experimental.pallas import tpu_sc as plsc

BATCH = 4
SEQ = 8192
N = BATCH * SEQ
D = 1024
NW = 32
ROWS_PER_W = N // NW
UNROLL = 16
NBLK = ROWS_PER_W // UNROLL
NCH = 16
CW = D // NCH


def _make_kernel():
    mesh = plsc.VectorSubcoreMesh(core_axis_name="c", subcore_axis_name="s")

    @functools.partial(
        pl.kernel,
        mesh=mesh,
        out_type=jax.ShapeDtypeStruct((N * D,), jnp.float32),
        scratch_types=[
            pltpu.VMEM((ROWS_PER_W,), jnp.int32),
            pltpu.VMEM((2 * D,), jnp.float32),
            pltpu.SemaphoreType.DMA,
        ],
    )
    def k(ids_hbm, table_hbm, out_hbm, idx_v, tab_v, sem):
        wid = lax.axis_index("s") * 2 + lax.axis_index("c")
        base = wid * ROWS_PER_W
        pltpu.sync_copy(ids_hbm.at[pl.ds(base, ROWS_PER_W)], idx_v)
        pltpu.sync_copy(table_hbm, tab_v)

        def body(blk, _):
            r0 = blk * UNROLL
            v = idx_v[pl.ds(r0, 16)]
            for j in range(UNROLL):
                for c in range(NCH):
                    src = tab_v.at[pl.ds(v[j] * D + c * CW, CW)]
                    dst = out_hbm.at[pl.ds((base + r0 + j) * D + c * CW, CW)]
                    pltpu.async_copy(src, dst, sem)
            return _

        lax.fori_loop(0, NBLK, body, None)
        pltpu.make_async_copy(
            out_hbm.at[pl.ds(base * D, ROWS_PER_W * D)],
            out_hbm.at[pl.ds(base * D, ROWS_PER_W * D)],
            sem,
        ).wait()

    return k


_k = _make_kernel()


def kernel(token_type_ids, table):
    ids_flat = token_type_ids.reshape(-1).astype(jnp.int32)
    out = _k(ids_flat, table.reshape(-1).astype(jnp.float32))
    return out.reshape(BATCH, SEQ, D)


# speedup vs baseline: 3.0496x; 3.0496x over previous
import functools

import jax
import jax.numpy as jnp
from jax import lax
from jax.experimental import pallas as pl
from jax.experimental.pallas import tpu as pltpu
from jax.experimental.pallas import tpu_sc as plsc

BATCH = 4
SEQ = 8192
N = BATCH * SEQ
D = 1024
NW = 32
ROWS_PER_W = N // NW
UNROLL = 16
NBLK = ROWS_PER_W // UNROLL


def _make_kernel():
    mesh = plsc.VectorSubcoreMesh(core_axis_name="c", subcore_axis_name="s")

    @functools.partial(
        pl.kernel,
        mesh=mesh,
        out_type=jax.ShapeDtypeStruct((N, D), jnp.float32),
        scratch_types=[
            pltpu.VMEM((ROWS_PER_W,), jnp.int32),
            pltpu.VMEM((2, D), jnp.float32),
            pltpu.SemaphoreType.DMA,
        ],
    )
    def k(ids_hbm, table_hbm, out_hbm, idx_v, tab_v, sem):
        wid = lax.axis_index("s") * 2 + lax.axis_index("c")
        base = wid * ROWS_PER_W
        pltpu.sync_copy(ids_hbm.at[pl.ds(base, ROWS_PER_W)], idx_v)
        pltpu.sync_copy(table_hbm, tab_v)

        def body(blk, _):
            r0 = blk * UNROLL
            v = idx_v[pl.ds(r0, 16)]
            for j in range(UNROLL):
                t = v[j]
                pltpu.async_copy(tab_v.at[t], out_hbm.at[base + r0 + j], sem)
            return _

        lax.fori_loop(0, NBLK, body, None)
        pltpu.make_async_copy(
            out_hbm.at[pl.ds(base, ROWS_PER_W)],
            out_hbm.at[pl.ds(base, ROWS_PER_W)],
            sem,
        ).wait()

    return k


_k = _make_kernel()


def kernel(token_type_ids, table):
    ids_flat = token_type_ids.reshape(-1).astype(jnp.int32)
    out = _k(ids_flat, table.astype(jnp.float32))
    return out.reshape(BATCH, SEQ, D)


# speedup vs baseline: 3.1600x; 1.0362x over previous
import functools

import jax
import jax.numpy as jnp
from jax import lax
from jax.experimental import pallas as pl
from jax.experimental.pallas import tpu as pltpu
from jax.experimental.pallas import tpu_sc as plsc

BATCH = 4
SEQ = 8192
N = BATCH * SEQ
D = 1024
NW = 32
ROWS_PER_W = N // NW
UNROLL = 16
NBLK = ROWS_PER_W // UNROLL
LOCAL_JS = tuple(j for j in range(UNROLL) if j % 3 == 2)
N_LOCAL = NBLK * len(LOCAL_JS)
N_STREAM = ROWS_PER_W - N_LOCAL


def _make_kernel():
    mesh = plsc.VectorSubcoreMesh(core_axis_name="c", subcore_axis_name="s")

    @functools.partial(
        pl.kernel,
        mesh=mesh,
        out_type=jax.ShapeDtypeStruct((N, D), jnp.float32),
        scratch_types=[
            pltpu.VMEM((ROWS_PER_W,), jnp.int32),
            pltpu.VMEM((2, D), jnp.float32),
            pltpu.VMEM_SHARED((2, D), jnp.float32),
            pltpu.SemaphoreType.DMA,
            pltpu.SemaphoreType.DMA,
        ],
    )
    def k(ids_hbm, table_hbm, out_hbm, idx_v, tab_v, tab_sp, sem, lsem):
        sid = lax.axis_index("s")
        wid = sid * 2 + lax.axis_index("c")
        base = wid * ROWS_PER_W
        pltpu.sync_copy(ids_hbm.at[pl.ds(base, ROWS_PER_W)], idx_v)
        pltpu.sync_copy(table_hbm, tab_v)

        @pl.when(sid == 0)
        def _stage_shared():
            pltpu.sync_copy(table_hbm, tab_sp)

        plsc.subcore_barrier()

        def body(blk, _):
            r0 = blk * UNROLL
            v = idx_v[pl.ds(r0, 16)]
            for j in range(UNROLL):
                t = v[j]
                if j in LOCAL_JS:
                    pltpu.async_copy(tab_sp.at[t], out_hbm.at[base + r0 + j],
                                     lsem)
                else:
                    pltpu.async_copy(tab_v.at[t], out_hbm.at[base + r0 + j],
                                     sem)
            return _

        lax.fori_loop(0, NBLK, body, None)
        pltpu.make_async_copy(
            out_hbm.at[pl.ds(base, N_STREAM)],
            out_hbm.at[pl.ds(base, N_STREAM)],
            sem,
        ).wait()
        pltpu.make_async_copy(
            out_hbm.at[pl.ds(base, N_LOCAL)],
            out_hbm.at[pl.ds(base, N_LOCAL)],
            lsem,
        ).wait()

    return k


_k = _make_kernel()


def kernel(token_type_ids, table):
    ids_flat = token_type_ids.reshape(-1).astype(jnp.int32)
    out = _k(ids_flat, table.astype(jnp.float32))
    return out.reshape(BATCH, SEQ, D)


# speedup vs baseline: 3.2106x; 1.0160x over previous
import functools

import jax
import jax.numpy as jnp
from jax import lax
from jax.experimental import pallas as pl
from jax.experimental.pallas import tpu as pltpu
from jax.experimental.pallas import tpu_sc as plsc

BATCH = 4
SEQ = 8192
N = BATCH * SEQ
D = 1024
NW = 32
ROWS_PER_W = N // NW
UNROLL = 16
NBLK = ROWS_PER_W // UNROLL
LOCAL_JS = tuple(j for j in range(UNROLL) if j % 4 == 3)
N_LOCAL = NBLK * len(LOCAL_JS)
N_STREAM = ROWS_PER_W - N_LOCAL


def _make_kernel():
    mesh = plsc.VectorSubcoreMesh(core_axis_name="c", subcore_axis_name="s")

    @functools.partial(
        pl.kernel,
        mesh=mesh,
        out_type=jax.ShapeDtypeStruct((BATCH, SEQ, D), jnp.float32),
        scratch_types=[
            pltpu.VMEM((ROWS_PER_W,), jnp.int32),
            pltpu.VMEM((2, D), jnp.float32),
            pltpu.VMEM_SHARED((2, D), jnp.float32),
            pltpu.SemaphoreType.DMA,
            pltpu.SemaphoreType.DMA,
        ],
    )
    def k(ids_hbm, table_hbm, out_hbm, idx_v, tab_v, tab_sp, sem, lsem):
        sid = lax.axis_index("s")
        wid = sid * 2 + lax.axis_index("c")
        b = wid // (SEQ // ROWS_PER_W)
        s0 = (wid % (SEQ // ROWS_PER_W)) * ROWS_PER_W
        cp_idx = pltpu.async_copy(ids_hbm.at[b, pl.ds(s0, ROWS_PER_W)],
                                  idx_v, sem)
        cp_tab = pltpu.async_copy(table_hbm, tab_v, sem)

        @pl.when(sid == 0)
        def _stage_shared():
            pltpu.async_copy(table_hbm, tab_sp, lsem).wait()

        cp_idx.wait()
        cp_tab.wait()
        plsc.subcore_barrier()

        def body(blk, _):
            r0 = blk * UNROLL
            v = idx_v[pl.ds(r0, 16)]
            for j in range(UNROLL):
                t = v[j]
                if j in LOCAL_JS:
                    pltpu.async_copy(tab_sp.at[t],
                                     out_hbm.at[b, s0 + r0 + j], lsem)
                else:
                    pltpu.async_copy(tab_v.at[t],
                                     out_hbm.at[b, s0 + r0 + j], sem)
            return _

        lax.fori_loop(0, NBLK, body, None)
        pltpu.make_async_copy(
            out_hbm.at[b, pl.ds(s0, N_STREAM)],
            out_hbm.at[b, pl.ds(s0, N_STREAM)],
            sem,
        ).wait()
        pltpu.make_async_copy(
            out_hbm.at[b, pl.ds(s0, N_LOCAL)],
            out_hbm.at[b, pl.ds(s0, N_LOCAL)],
            lsem,
        ).wait()

    return k


_k = _make_kernel()


def kernel(token_type_ids, table):
    return _k(token_type_ids.astype(jnp.int32), table.astype(jnp.float32))
